# Initial kernel scaffold; baseline (speedup 1.0000x reference)
import jax
import jax.numpy as jnp
from jax.experimental import pallas as pl


def kernel(image_feature, memory, W_fuse, b_fuse, W_dw, b_dw):
    raise NotImplementedError("write your pallas kernel here")


# speedup vs baseline: 19.9166x; 19.9166x over previous
import functools

import jax
import jax.numpy as jnp
from jax.experimental import pallas as pl

_DIL = 2
_NEG_INF = float("-inf")


def _fused_body(H, W, x_ref, mem_ref, memT_ref, wfa_ref, wfb_ref, bf_ref,
                taps_ref, bdw_ref, out_ref):
    P, C = x_ref.shape[1], x_ref.shape[2]
    M = mem_ref.shape[0]
    x = x_ref[0]
    mem = mem_ref[...]
    memT = memT_ref[...]

    ig = jnp.mean(x, axis=0, keepdims=True)
    sg = jnp.dot(ig, memT, preferred_element_type=jnp.float32)
    sg = sg - jnp.max(sg, axis=1, keepdims=True)
    eg = jnp.exp(sg)
    smg = eg / jnp.sum(eg, axis=1, keepdims=True)
    mr = jnp.dot(smg, mem, preferred_element_type=jnp.float32) + ig
    gate = 1.0 / (1.0 + jnp.exp(-mr))
    gx = x * gate

    S = jnp.dot(x, memT, preferred_element_type=jnp.float32)
    col = jax.lax.broadcasted_iota(jnp.int32, (P, M), 1)
    v1 = jnp.max(S, axis=1, keepdims=True)
    i1 = jnp.min(jnp.where(S == v1, col, M), axis=1, keepdims=True)
    S2 = jnp.where(col == i1, _NEG_INF, S)
    v2 = jnp.max(S2, axis=1, keepdims=True)
    i2 = jnp.min(jnp.where(S2 == v2, col, M), axis=1, keepdims=True)
    e2 = jnp.exp(v2 - v1)
    a1 = 1.0 / (1.0 + e2)
    a2 = 1.0 - a1
    wattn = jnp.where(col == i1, a1, 0.0) + jnp.where(col == i2, a2, 0.0)
    mf = jnp.dot(wattn, mem, preferred_element_type=jnp.float32)

    Y = (jnp.dot(gx, wfa_ref[...], preferred_element_type=jnp.float32)
         + jnp.dot(mf, wfb_ref[...], preferred_element_type=jnp.float32)
         + bf_ref[...])
    Y = jnp.where(Y > 0, Y, 0.2 * Y)

    Yh = Y.reshape(H, W, C)

    def shift(a, axis, d):
        if d == 0:
            return a
        zshape = list(a.shape)
        zshape[axis] = abs(d)
        z = jnp.zeros(zshape, a.dtype)
        n = a.shape[axis]
        if d > 0:
            body = jax.lax.slice_in_dim(a, d, n, axis=axis)
            return jnp.concatenate([body, z], axis=axis)
        body = jax.lax.slice_in_dim(a, 0, n + d, axis=axis)
        return jnp.concatenate([z, body], axis=axis)

    acc = jnp.zeros((H, W, C), jnp.float32)
    k = 0
    for kh in range(3):
        for kw in range(3):
            dh = (kh - 1) * _DIL
            dw = (kw - 1) * _DIL
            win = shift(shift(Yh, 0, dh), 1, dw)
            acc = acc + win * taps_ref[k, :][None, None, :]
            k += 1
    acc = acc + bdw_ref[0, :][None, None, :]
    out_ref[0] = jnp.where(acc > 0, acc, 0.2 * acc)


def kernel(image_feature, memory, W_fuse, b_fuse, W_dw, b_dw):
    B, C, H, W = image_feature.shape
    M = memory.shape[0]
    P = H * W
    x_pc = image_feature.reshape(B, C, P).transpose(0, 2, 1)
    memT = memory.T
    wfa = W_fuse[:, :C].T
    wfb = W_fuse[:, C:].T
    taps = W_dw[:, 0, :, :].reshape(C, 9).T
    bf = b_fuse.reshape(1, C)
    bdw = b_dw.reshape(1, C)

    out = pl.pallas_call(
        functools.partial(_fused_body, H, W),
        grid=(B,),
        in_specs=[
            pl.BlockSpec((1, P, C), lambda b: (b, 0, 0)),
            pl.BlockSpec((M, C), lambda b: (0, 0)),
            pl.BlockSpec((C, M), lambda b: (0, 0)),
            pl.BlockSpec((C, C), lambda b: (0, 0)),
            pl.BlockSpec((C, C), lambda b: (0, 0)),
            pl.BlockSpec((1, C), lambda b: (0, 0)),
            pl.BlockSpec((9, C), lambda b: (0, 0)),
            pl.BlockSpec((1, C), lambda b: (0, 0)),
        ],
        out_specs=pl.BlockSpec((1, H, W, C), lambda b: (b, 0, 0, 0)),
        out_shape=jax.ShapeDtypeStruct((B, H, W, C), jnp.float32),
    )(x_pc, memory, memT, wfa, wfb, bf, taps, bdw)
    return out.transpose(0, 3, 1, 2)
